# Initial kernel scaffold; baseline (speedup 1.0000x reference)
import jax
import jax.numpy as jnp
from jax.experimental import pallas as pl


def kernel(x, edge_index, W1, b1, W2, b2):
    raise NotImplementedError("write your pallas kernel here")


# speedup vs baseline: 12.2783x; 12.2783x over previous
import functools

import jax
import jax.numpy as jnp
from jax import lax
from jax.experimental import pallas as pl
from jax.experimental.pallas import tpu as pltpu
from jax.experimental.pallas import tpu_sc as plsc

N = 10000
NPAD = 10240
E = 320000
D = 128
OUT = 100
NC, NS = 2, 16
NW = NC * NS
EPT = E // NW
K = 80
NCHUNK = EPT // K
RPT = NPAD // NS
DEGW = 128

_mesh = plsc.VectorSubcoreMesh(
    core_axis_name="c", subcore_axis_name="s", num_cores=NC, num_subcores=NS
)


def _fill_rows(ref, rows, width, value):
  vec = jnp.full((16,), value, jnp.float32)

  def body(i, c):
    for q in range(width // 16):
      ref[i, pl.ds(q * 16, 16)] = vec
    return c

  lax.fori_loop(0, rows, body, 0)


@functools.partial(
    pl.kernel,
    out_type=jax.ShapeDtypeStruct((NC, NPAD, DEGW), jnp.float32),
    mesh=_mesh,
    scratch_types=[
        pltpu.VMEM((K,), jnp.int32),
        pltpu.VMEM((K, DEGW), jnp.float32),
        pltpu.VMEM_SHARED((NPAD, DEGW), jnp.float32),
    ],
)
def _deg_kernel(dst_hbm, out_hbm, idx_v, ones_v, hist):
  cid = lax.axis_index("c")
  sid = lax.axis_index("s")
  wid = cid * NS + sid

  _fill_rows(ones_v, K, DEGW, 0.0)
  for z in range(RPT // K):
    pltpu.sync_copy(ones_v, hist.at[pl.ds(sid * RPT + z * K, K)])
  _fill_rows(ones_v, K, DEGW, 1.0)
  plsc.subcore_barrier()

  base = wid * EPT

  def chunk(j, c):
    pltpu.sync_copy(dst_hbm.at[pl.ds(base + j * K, K)], idx_v)
    pltpu.sync_copy(ones_v, hist.at[idx_v], add=True)
    return c

  lax.fori_loop(0, NCHUNK, chunk, 0)
  plsc.subcore_barrier()
  pltpu.sync_copy(
      hist.at[pl.ds(sid * RPT, RPT)],
      out_hbm.at[cid].at[pl.ds(sid * RPT, RPT)],
  )


@functools.partial(
    pl.kernel,
    out_type=jax.ShapeDtypeStruct((NC, NPAD, D), jnp.float32),
    mesh=_mesh,
    scratch_types=[
        pltpu.VMEM((K,), jnp.int32),
        pltpu.VMEM((K,), jnp.int32),
        pltpu.VMEM((K, D), jnp.float32),
        pltpu.VMEM((K, D), jnp.float32),
        pltpu.VMEM_SHARED((NPAD, D), jnp.float32),
        pltpu.SemaphoreType.DMA,
    ],
)
def _acc_kernel(y_hbm, src_hbm, dst_hbm, out_hbm, src_v, dst_v, rows_v,
                zeros_v, acc, sem):
  cid = lax.axis_index("c")
  sid = lax.axis_index("s")
  wid = cid * NS + sid

  _fill_rows(zeros_v, K, D, 0.0)
  for z in range(RPT // K):
    pltpu.sync_copy(zeros_v, acc.at[pl.ds(sid * RPT + z * K, K)])
  plsc.subcore_barrier()

  base = wid * EPT

  def chunk(j, c):
    pltpu.sync_copy(src_hbm.at[pl.ds(base + j * K, K)], src_v)
    pltpu.sync_copy(dst_hbm.at[pl.ds(base + j * K, K)], dst_v)
    pltpu.async_copy(y_hbm.at[src_v], rows_v, sem).wait()
    pltpu.sync_copy(rows_v, acc.at[dst_v], add=True)
    return c

  lax.fori_loop(0, NCHUNK, chunk, 0)
  plsc.subcore_barrier()
  pltpu.sync_copy(
      acc.at[pl.ds(sid * RPT, RPT)],
      out_hbm.at[cid].at[pl.ds(sid * RPT, RPT)],
  )


_GRID = NPAD // 1024


def _tcA_body(x_ref, w_ref, da_ref, db_ref, y_ref, dinv_ref):
  deg = da_ref[:, 0:1] + db_ref[:, 0:1] + 1.0
  dinv = lax.rsqrt(deg)
  y_ref[...] = (
      jnp.dot(x_ref[...], w_ref[...], preferred_element_type=jnp.float32)
      * dinv
  )
  dinv_ref[...] = dinv


def _tcB_body(a0_ref, a1_ref, y1_ref, dinv_ref, b1_ref, w2_ref, y2_ref):
  dinv = dinv_ref[...]
  h = dinv * (a0_ref[...] + a1_ref[...] + y1_ref[...]) + b1_ref[...]
  h = jnp.maximum(h, 0.0)
  y2_ref[...] = (
      jnp.dot(h, w2_ref[...], preferred_element_type=jnp.float32) * dinv
  )


def _tcC_body(a0_ref, a1_ref, y2_ref, dinv_ref, b2_ref, out_ref):
  t = dinv_ref[...] * (a0_ref[...] + a1_ref[...] + y2_ref[...]) + b2_ref[...]
  out_ref[...] = t[:, :OUT]


def _row_spec(w):
  return pl.BlockSpec((1024, w), lambda i: (i, 0))


def _full_spec(shape):
  return pl.BlockSpec(shape, lambda i: tuple(0 for _ in shape))


def kernel(x, edge_index, W1, b1, W2, b2):
  src = edge_index[0].astype(jnp.int32)
  dst = edge_index[1].astype(jnp.int32)
  xp = jnp.pad(x, ((0, NPAD - N), (0, 0)))
  w2p = jnp.pad(W2, ((0, 0), (0, D - OUT)))
  b1r = b1.reshape(1, D)
  b2r = jnp.pad(b2, (0, D - OUT)).reshape(1, D)

  deg_parts = _deg_kernel(dst)

  y1, dinv = pl.pallas_call(
      _tcA_body,
      grid=(_GRID,),
      in_specs=[
          _row_spec(D),
          _full_spec((D, D)),
          _row_spec(DEGW),
          _row_spec(DEGW),
      ],
      out_specs=[_row_spec(D), _row_spec(1)],
      out_shape=[
          jax.ShapeDtypeStruct((NPAD, D), jnp.float32),
          jax.ShapeDtypeStruct((NPAD, 1), jnp.float32),
      ],
  )(xp, W1, deg_parts[0], deg_parts[1])

  acc1 = _acc_kernel(y1, src, dst)

  y2 = pl.pallas_call(
      _tcB_body,
      grid=(_GRID,),
      in_specs=[
          _row_spec(D),
          _row_spec(D),
          _row_spec(D),
          _row_spec(1),
          _full_spec((1, D)),
          _full_spec((D, D)),
      ],
      out_specs=_row_spec(D),
      out_shape=jax.ShapeDtypeStruct((NPAD, D), jnp.float32),
  )(acc1[0], acc1[1], y1, dinv, b1r, w2p)

  acc2 = _acc_kernel(y2, src, dst)

  outp = pl.pallas_call(
      _tcC_body,
      grid=(_GRID,),
      in_specs=[
          _row_spec(D),
          _row_spec(D),
          _row_spec(D),
          _row_spec(1),
          _full_spec((1, D)),
      ],
      out_specs=_row_spec(OUT),
      out_shape=jax.ShapeDtypeStruct((NPAD, OUT), jnp.float32),
  )(acc2[0], acc2[1], y2, dinv, b2r)

  return outp[:N]
